# baseline (speedup 1.0000x reference)
import jax
import jax.numpy as jnp
from jax import lax
from jax.experimental import pallas as pl
from jax.experimental.pallas import tpu as pltpu
from jax.experimental.pallas import tpu_sc as plsc

N = 1000
B = 16384
Q = 2 * B
NC = 2
NS = 16
NW = NC * NS
BPW = B // NW
CHUNK = 128
NCH = BPW // CHUNK
NV = BPW // 16


def _tc_body(wmA_ref, cab_ref, wmB_ref, cba_ref, tA_ref, tB_ref):
    def t_for(wm, c):
        m = jnp.max(c, axis=1)
        lse = jnp.log(jnp.sum(jnp.exp(c - m[:, None]), axis=1)) + m
        mm = jnp.max(wm)
        lse_m = jnp.log(jnp.sum(jnp.exp(wm - mm))) + mm
        return wm - lse_m - lse

    tA_ref[:] = t_for(wmA_ref[:], cab_ref[:])
    tB_ref[:] = t_for(wmB_ref[:], cba_ref[:])


_tc_call = pl.pallas_call(
    _tc_body,
    out_shape=(
        jax.ShapeDtypeStruct((N,), jnp.float32),
        jax.ShapeDtypeStruct((N,), jnp.float32),
    ),
)


def _sc_body(q_hbm, wab_hbm, wba_hbm,
             outA_hbm, outB_hbm,
             a_v, b_v, idxE, idxO, idxA, idxB, gA, gB,
             accA_v, accB_v, sem):
    wid = lax.axis_index("s") * NC + lax.axis_index("c")
    base = wid * BPW
    lane = jnp.arange(16, dtype=jnp.int32)
    for j in range(NV):
        e = 2 * (base + 16 * j) + 2 * lane
        idxE[j // 8, pl.ds(16 * (j % 8), 16)] = e
        idxO[j // 8, pl.ds(16 * (j % 8), 16)] = e + 1
    in_cp = []
    for c in range(NCH):
        in_cp.append(pltpu.async_copy(
            q_hbm.at[idxE.at[c]], a_v.at[pl.ds(CHUNK * c, CHUNK)], sem))
        in_cp.append(pltpu.async_copy(
            q_hbm.at[idxO.at[c]], b_v.at[pl.ds(CHUNK * c, CHUNK)], sem))
    for cp in in_cp:
        cp.wait()
    for j in range(NV):
        a16 = a_v[pl.ds(16 * j, 16)]
        b16 = b_v[pl.ds(16 * j, 16)]
        idxA[j // 8, pl.ds(16 * (j % 8), 16)] = a16 * N + b16
        idxB[j // 8, pl.ds(16 * (j % 8), 16)] = b16 * N + a16
    copies = []
    for c in range(NCH):
        copies.append(pltpu.async_copy(wab_hbm.at[idxA.at[c]], gA.at[c], sem))
        copies.append(pltpu.async_copy(wba_hbm.at[idxB.at[c]], gB.at[c], sem))
    for cp in copies:
        cp.wait()
    accA = jnp.zeros((16,), jnp.float32)
    accB = jnp.zeros((16,), jnp.float32)
    for j in range(NV):
        r, s = j // 8, pl.ds(16 * (j % 8), 16)
        accA = accA + gA[r, s]
        accB = accB + gB[r, s]
    accA_v[:] = accA
    accB_v[:] = accB
    pltpu.sync_copy(accA_v, outA_hbm.at[wid])
    pltpu.sync_copy(accB_v, outB_hbm.at[wid])


_sc_call = pl.kernel(
    _sc_body,
    out_type=(
        jax.ShapeDtypeStruct((NW, 16), jnp.float32),
        jax.ShapeDtypeStruct((NW, 16), jnp.float32),
    ),
    mesh=plsc.VectorSubcoreMesh(core_axis_name="c", subcore_axis_name="s"),
    scratch_types=(
        pltpu.VMEM((BPW,), jnp.int32),
        pltpu.VMEM((BPW,), jnp.int32),
        pltpu.VMEM((NCH, CHUNK), jnp.int32),
        pltpu.VMEM((NCH, CHUNK), jnp.int32),
        pltpu.VMEM((NCH, CHUNK), jnp.int32),
        pltpu.VMEM((NCH, CHUNK), jnp.int32),
        pltpu.VMEM((NCH, CHUNK), jnp.float32),
        pltpu.VMEM((NCH, CHUNK), jnp.float32),
        pltpu.VMEM((16,), jnp.float32),
        pltpu.VMEM((16,), jnp.float32),
        pltpu.SemaphoreType.DMA,
    ),
)


def _combine_body(w_ref, q_ref, tA_ref, tB_ref, pA_ref, pB_ref, out_ref):
    v = q_ref[:]
    qh = jnp.right_shift(v, 5)
    sl = jnp.bitwise_and(v, 31)
    lvl = lax.broadcasted_iota(jnp.int32, (32, Q), 0)
    pos = lax.broadcasted_iota(jnp.int32, (32, Q), 1)
    even = (jnp.bitwise_and(pos, 1) == 0).astype(jnp.float32)
    oh_q = (qh[None, :] == lvl).astype(jnp.float32)
    oh_s = (sl[None, :] == lvl).astype(jnp.float32)
    ohqA = oh_q * even
    ohqB = oh_q - ohqA
    cntA = lax.dot_general(ohqA, oh_s, (((1,), (1,)), ((), ())),
                           preferred_element_type=jnp.float32)
    cntB = lax.dot_general(ohqB, oh_s, (((1,), (1,)), ((), ())),
                           preferred_element_type=jnp.float32)

    zpad = jnp.zeros((24,), jnp.float32)
    tpadA = jnp.concatenate([tA_ref[:], zpad])
    tpadB = jnp.concatenate([tB_ref[:], zpad])
    accA = jnp.zeros((32,), jnp.float32)
    accB = jnp.zeros((32,), jnp.float32)
    for qq in range(32):
        accA = accA + cntA[qq, :] * tpadA[32 * qq:32 * qq + 32]
        accB = accB + cntB[qq, :] * tpadB[32 * qq:32 * qq + 32]
    S_AB = jnp.sum(accA) + jnp.sum(pA_ref[:])
    S_BA = jnp.sum(accB) + jnp.sum(pB_ref[:])
    wv = w_ref[:, :]
    la = -jnp.log(1.0 + jnp.exp(-wv))
    l1a = -jnp.log(1.0 + jnp.exp(wv))
    x = la + S_AB
    y = l1a + S_BA
    m = jnp.maximum(x, y)
    out_ref[:, :] = m + jnp.log(jnp.exp(x - m) + jnp.exp(y - m))


_combine_call = pl.pallas_call(
    _combine_body,
    out_shape=jax.ShapeDtypeStruct((1, 1), jnp.float32),
)


def kernel(inputs, w, w_mA, w_cAB, w_mB, w_cBA):
    q = inputs.reshape(-1)
    tA, tB = _tc_call(w_mA, w_cAB, w_mB, w_cBA)
    outA, outB = _sc_call(q, w_cAB.reshape(-1), w_cBA.reshape(-1))
    res = _combine_call(jnp.reshape(w, (1, 1)), q, tA, tB, outA, outB)
    return jnp.reshape(res, ())


# speedup vs baseline: 1.2555x; 1.2555x over previous
import jax
import jax.numpy as jnp
from jax import lax
from jax.experimental import pallas as pl
from jax.experimental.pallas import tpu as pltpu
from jax.experimental.pallas import tpu_sc as plsc

N = 1000
B = 16384
NC = 2
NS = 16
NW = NC * NS
BPW = B // NW
CHUNK = 128
NCH = BPW // CHUNK
NV = BPW // 16
CB = 2048
CSTEPS = B // CB


def _tc_body(wmA_ref, cab_ref, wmB_ref, cba_ref, tA_ref, tB_ref):
    def t_for(wm, c):
        m = jnp.max(c, axis=1)
        lse = jnp.log(jnp.sum(jnp.exp(c - m[:, None]), axis=1)) + m
        mm = jnp.max(wm)
        lse_m = jnp.log(jnp.sum(jnp.exp(wm - mm))) + mm
        return wm - lse_m - lse

    tA_ref[:] = t_for(wmA_ref[:], cab_ref[:])
    tB_ref[:] = t_for(wmB_ref[:], cba_ref[:])


_tc_call = pl.pallas_call(
    _tc_body,
    out_shape=(
        jax.ShapeDtypeStruct((N,), jnp.float32),
        jax.ShapeDtypeStruct((N,), jnp.float32),
    ),
)


def _sc_body(a_hbm, b_hbm, wab_hbm, wba_hbm,
             outA_hbm, outB_hbm,
             a_v, b_v, idxA, idxB, gA, gB,
             accA_v, accB_v, sem):
    wid = lax.axis_index("s") * NC + lax.axis_index("c")
    base = wid * BPW
    pltpu.sync_copy(a_hbm.at[pl.ds(base, BPW)], a_v)
    pltpu.sync_copy(b_hbm.at[pl.ds(base, BPW)], b_v)
    for j in range(NV):
        a16 = a_v[pl.ds(16 * j, 16)]
        b16 = b_v[pl.ds(16 * j, 16)]
        idxA[j // 8, pl.ds(16 * (j % 8), 16)] = a16 * N + b16
        idxB[j // 8, pl.ds(16 * (j % 8), 16)] = b16 * N + a16
    copies = []
    for c in range(NCH):
        copies.append(pltpu.async_copy(wab_hbm.at[idxA.at[c]], gA.at[c], sem))
        copies.append(pltpu.async_copy(wba_hbm.at[idxB.at[c]], gB.at[c], sem))
    for cp in copies:
        cp.wait()
    accA = jnp.zeros((16,), jnp.float32)
    accB = jnp.zeros((16,), jnp.float32)
    for j in range(NV):
        r, s = j // 8, pl.ds(16 * (j % 8), 16)
        accA = accA + gA[r, s]
        accB = accB + gB[r, s]
    accA_v[:] = accA
    accB_v[:] = accB
    pltpu.sync_copy(accA_v, outA_hbm.at[wid])
    pltpu.sync_copy(accB_v, outB_hbm.at[wid])


_sc_call = pl.kernel(
    _sc_body,
    out_type=(
        jax.ShapeDtypeStruct((NW, 16), jnp.float32),
        jax.ShapeDtypeStruct((NW, 16), jnp.float32),
    ),
    mesh=plsc.VectorSubcoreMesh(core_axis_name="c", subcore_axis_name="s"),
    scratch_types=(
        pltpu.VMEM((BPW,), jnp.int32),
        pltpu.VMEM((BPW,), jnp.int32),
        pltpu.VMEM((NCH, CHUNK), jnp.int32),
        pltpu.VMEM((NCH, CHUNK), jnp.int32),
        pltpu.VMEM((NCH, CHUNK), jnp.float32),
        pltpu.VMEM((NCH, CHUNK), jnp.float32),
        pltpu.VMEM((16,), jnp.float32),
        pltpu.VMEM((16,), jnp.float32),
        pltpu.SemaphoreType.DMA,
    ),
)


def _count_dot(v, tpad):
    q = jnp.right_shift(v, 5)
    s = jnp.bitwise_and(v, 31)
    lvl = lax.broadcasted_iota(jnp.int32, (32, B), 0)
    oh_q = (q[None, :] == lvl).astype(jnp.float32)
    oh_s = (s[None, :] == lvl).astype(jnp.float32)
    cnt = lax.dot_general(oh_q, oh_s, (((1,), (1,)), ((), ())),
                          preferred_element_type=jnp.float32)
    acc = jnp.zeros((32,), jnp.float32)
    for qq in range(32):
        acc = acc + cnt[qq, :] * tpad[32 * qq:32 * qq + 32]
    return jnp.sum(acc)


def _combine_body(w_ref, a_ref, b_ref, wmA_ref, cab_ref, wmB_ref, cba_ref,
                  pA_ref, pB_ref, out_ref):
    def t_for(wm, c):
        m = jnp.max(c, axis=1)
        lse = jnp.log(jnp.sum(jnp.exp(c - m[:, None]), axis=1)) + m
        mm = jnp.max(wm)
        lse_m = jnp.log(jnp.sum(jnp.exp(wm - mm))) + mm
        return wm - lse_m - lse

    zpad = jnp.zeros((24,), jnp.float32)
    tpadA = jnp.concatenate([t_for(wmA_ref[:], cab_ref[:]), zpad])
    tpadB = jnp.concatenate([t_for(wmB_ref[:], cba_ref[:]), zpad])
    S_AB = _count_dot(a_ref[:], tpadA) + jnp.sum(pA_ref[:])
    S_BA = _count_dot(b_ref[:], tpadB) + jnp.sum(pB_ref[:])
    wv = w_ref[:, :]
    la = -jnp.log(1.0 + jnp.exp(-wv))
    l1a = -jnp.log(1.0 + jnp.exp(wv))
    x = la + S_AB
    y = l1a + S_BA
    m = jnp.maximum(x, y)
    out_ref[:, :] = m + jnp.log(jnp.exp(x - m) + jnp.exp(y - m))


_combine_call = pl.pallas_call(
    _combine_body,
    out_shape=jax.ShapeDtypeStruct((1, 1), jnp.float32),
)


def kernel(inputs, w, w_mA, w_cAB, w_mB, w_cBA):
    a = inputs[:, 0]
    b = inputs[:, 1]
    outA, outB = _sc_call(a, b, w_cAB.reshape(-1), w_cBA.reshape(-1))
    res = _combine_call(jnp.reshape(w, (1, 1)), a, b, w_mA, w_cAB,
                        w_mB, w_cBA, outA, outB)
    return jnp.reshape(res, ())


# speedup vs baseline: 1.3530x; 1.0777x over previous
import jax
import jax.numpy as jnp
from jax import lax
from jax.experimental import pallas as pl
from jax.experimental.pallas import tpu as pltpu
from jax.experimental.pallas import tpu_sc as plsc

N = 1000
B = 16384
NC = 2
NS = 16
NW = NC * NS
BPW = B // NW
CHUNK = 128
NCH = BPW // CHUNK
NV = BPW // 16
CB = 2048
CSTEPS = B // CB


def _tc_body(wmA_ref, cab_ref, wmB_ref, cba_ref, tA_ref, tB_ref):
    def t_for(wm, c):
        m = jnp.max(c, axis=1)
        lse = jnp.log(jnp.sum(jnp.exp(c - m[:, None]), axis=1)) + m
        mm = jnp.max(wm)
        lse_m = jnp.log(jnp.sum(jnp.exp(wm - mm))) + mm
        return wm - lse_m - lse

    tA_ref[:] = t_for(wmA_ref[:], cab_ref[:])
    tB_ref[:] = t_for(wmB_ref[:], cba_ref[:])


_tc_call = pl.pallas_call(
    _tc_body,
    out_shape=(
        jax.ShapeDtypeStruct((N,), jnp.float32),
        jax.ShapeDtypeStruct((N,), jnp.float32),
    ),
)


def _sc_body(a_hbm, b_hbm, wab_hbm, wba_hbm,
             outA_hbm, outB_hbm,
             a_v, b_v, idxA, idxB, gA, gB,
             accA_v, accB_v, sem):
    wid = lax.axis_index("s") * NC + lax.axis_index("c")
    base = wid * BPW
    pltpu.sync_copy(a_hbm.at[pl.ds(base, BPW)], a_v)
    pltpu.sync_copy(b_hbm.at[pl.ds(base, BPW)], b_v)
    for j in range(NV):
        a16 = a_v[pl.ds(16 * j, 16)]
        b16 = b_v[pl.ds(16 * j, 16)]
        idxA[j // 8, pl.ds(16 * (j % 8), 16)] = a16 * N + b16
        idxB[j // 8, pl.ds(16 * (j % 8), 16)] = b16 * N + a16
    copies = []
    for c in range(NCH):
        copies.append(pltpu.async_copy(wab_hbm.at[idxA.at[c]], gA.at[c], sem))
        copies.append(pltpu.async_copy(wba_hbm.at[idxB.at[c]], gB.at[c], sem))
    for cp in copies:
        cp.wait()
    accA = jnp.zeros((16,), jnp.float32)
    accB = jnp.zeros((16,), jnp.float32)
    for j in range(NV):
        r, s = j // 8, pl.ds(16 * (j % 8), 16)
        accA = accA + gA[r, s]
        accB = accB + gB[r, s]
    accA_v[:] = accA
    accB_v[:] = accB
    pltpu.sync_copy(accA_v, outA_hbm.at[wid])
    pltpu.sync_copy(accB_v, outB_hbm.at[wid])


_sc_call = pl.kernel(
    _sc_body,
    out_type=(
        jax.ShapeDtypeStruct((NW, 16), jnp.float32),
        jax.ShapeDtypeStruct((NW, 16), jnp.float32),
    ),
    mesh=plsc.VectorSubcoreMesh(core_axis_name="c", subcore_axis_name="s"),
    scratch_types=(
        pltpu.VMEM((BPW,), jnp.int32),
        pltpu.VMEM((BPW,), jnp.int32),
        pltpu.VMEM((NCH, CHUNK), jnp.int32),
        pltpu.VMEM((NCH, CHUNK), jnp.int32),
        pltpu.VMEM((NCH, CHUNK), jnp.float32),
        pltpu.VMEM((NCH, CHUNK), jnp.float32),
        pltpu.VMEM((16,), jnp.float32),
        pltpu.VMEM((16,), jnp.float32),
        pltpu.SemaphoreType.DMA,
    ),
)


def _count_dot(v, tpad):
    q = jnp.right_shift(v, 5)
    s = jnp.bitwise_and(v, 31)
    lvl = lax.broadcasted_iota(jnp.int32, (32, B), 0)
    oh_q = (q[None, :] == lvl).astype(jnp.float32)
    oh_s = (s[None, :] == lvl).astype(jnp.float32)
    cnt = lax.dot_general(oh_q, oh_s, (((1,), (1,)), ((), ())),
                          preferred_element_type=jnp.float32)
    acc = jnp.zeros((32,), jnp.float32)
    for qq in range(32):
        acc = acc + cnt[qq, :] * tpad[32 * qq:32 * qq + 32]
    return jnp.sum(acc)


def _dense_body(a_ref, b_ref, wmA_ref, cab_ref, wmB_ref, cba_ref, d_ref):
    def t_for(wm, c):
        m = jnp.max(c, axis=1)
        lse = jnp.log(jnp.sum(jnp.exp(c - m[:, None]), axis=1)) + m
        mm = jnp.max(wm)
        lse_m = jnp.log(jnp.sum(jnp.exp(wm - mm))) + mm
        return wm - lse_m - lse

    zpad = jnp.zeros((24,), jnp.float32)
    tpadA = jnp.concatenate([t_for(wmA_ref[:], cab_ref[:]), zpad])
    tpadB = jnp.concatenate([t_for(wmB_ref[:], cba_ref[:]), zpad])
    d_ref[:, :] = jnp.stack(
        [_count_dot(a_ref[:], tpadA), _count_dot(b_ref[:], tpadB)]
    ).reshape(1, 2)


_dense_call = pl.pallas_call(
    _dense_body,
    out_shape=jax.ShapeDtypeStruct((1, 2), jnp.float32),
)


def _final_body(w_ref, d_ref, pA_ref, pB_ref, out_ref):
    S_AB = d_ref[0, 0] + jnp.sum(pA_ref[:])
    S_BA = d_ref[0, 1] + jnp.sum(pB_ref[:])
    wv = w_ref[:, :]
    la = -jnp.log(1.0 + jnp.exp(-wv))
    l1a = -jnp.log(1.0 + jnp.exp(wv))
    x = la + S_AB
    y = l1a + S_BA
    m = jnp.maximum(x, y)
    out_ref[:, :] = m + jnp.log(jnp.exp(x - m) + jnp.exp(y - m))


_final_call = pl.pallas_call(
    _final_body,
    out_shape=jax.ShapeDtypeStruct((1, 1), jnp.float32),
)


def kernel(inputs, w, w_mA, w_cAB, w_mB, w_cBA):
    a = inputs[:, 0]
    b = inputs[:, 1]
    outA, outB = _sc_call(a, b, w_cAB.reshape(-1), w_cBA.reshape(-1))
    dots = _dense_call(a, b, w_mA, w_cAB, w_mB, w_cBA)
    res = _final_call(jnp.reshape(w, (1, 1)), dots, outA, outB)
    return jnp.reshape(res, ())
